# baseline (speedup 1.0000x reference)
import functools

import jax
import jax.numpy as jnp
import numpy as np
from jax import lax
from jax.experimental import pallas as pl

N_NODES = 10000
N_EDGES = 160000
NUM_SPECIES = 10
NCH = 128
R_MAX = 5.0
NUM_BESSEL = 8
P_CUT = 5
AVG_NEI = 16.0
L_LIST = [0, 1, 1, 1, 2, 2, 2, 2, 2, 3, 3, 3, 3, 3, 3, 3]

_EDGE_BLK = 2000


def _silu(x):
    return x / (1.0 + jnp.exp(-x))


def _edge_kernel(ea_ref, w1a, w1b, w1c, w1d, w2a, w2b, w2c, w2d,
                 tpw1_ref, tpw2_ref, shk_ref):
    ea = ea_ref[...]
    r = ea[:, 0:1]
    x = ea[:, 1:2]
    y = ea[:, 2:3]
    z = ea[:, 3:4]
    keep = ea[:, 4:5]

    n = jnp.sqrt(x * x + y * y + z * z)
    inv = 1.0 / jnp.maximum(n, 1e-9)
    x = x * inv
    y = y * inv
    z = z * inv
    s3 = 3.0 ** 0.5
    s5 = 5.0 ** 0.5
    s15 = 15.0 ** 0.5
    one = jnp.ones_like(x)
    sh = [one,
          s3 * x, s3 * y, s3 * z,
          s15 * x * y, s15 * y * z, 0.5 * s5 * (3.0 * z * z - 1.0),
          s15 * x * z, 0.5 * s15 * (x * x - y * y),
          0.25 * np.sqrt(70.0) * y * (3.0 * x * x - y * y),
          np.sqrt(105.0) * x * y * z,
          0.25 * np.sqrt(42.0) * y * (5.0 * z * z - 1.0),
          0.5 * np.sqrt(7.0) * (5.0 * z * z * z - 3.0 * z),
          0.25 * np.sqrt(42.0) * x * (5.0 * z * z - 1.0),
          0.5 * np.sqrt(105.0) * z * (x * x - y * y),
          0.25 * np.sqrt(70.0) * x * (x * x - 3.0 * y * y)]
    shm = jnp.concatenate(sh, axis=1)
    shk_ref[...] = shm * (keep * (1.0 / AVG_NEI))

    k = lax.broadcasted_iota(jnp.int32, (1, NUM_BESSEL), 1).astype(jnp.float32) + 1.0
    rb = np.sqrt(2.0 / R_MAX) * jnp.sin(k * (np.pi / R_MAX) * r) / jnp.maximum(r, 1e-9)
    xx = r / R_MAX
    p = float(P_CUT)
    xp = xx * xx * xx * xx * xx
    f = (1.0 - 0.5 * (p + 1.0) * (p + 2.0) * xp
         + p * (p + 2.0) * xp * xx
         - 0.5 * p * (p + 1.0) * xp * xx * xx)
    f = f * (xx < 1.0).astype(jnp.float32)
    rf = rb * f

    for (wa, wb, wc, wd, out) in ((w1a, w1b, w1c, w1d, tpw1_ref),
                                  (w2a, w2b, w2c, w2d, tpw2_ref)):
        h = _silu(jnp.dot(rf, wa[...], preferred_element_type=jnp.float32))
        h = _silu(jnp.dot(h, wb[...], preferred_element_type=jnp.float32))
        h = _silu(jnp.dot(h, wc[...], preferred_element_type=jnp.float32))
        out[...] = jnp.dot(h, wd[...], preferred_element_type=jnp.float32)


def _edge_features(ea5, rmlp1, rmlp2):
    E = ea5.shape[0]
    grid = (E // _EDGE_BLK,)
    wspec = [pl.BlockSpec(w.shape, lambda i: (0,) * w.ndim) for w in rmlp1 + rmlp2]
    return pl.pallas_call(
        _edge_kernel,
        grid=grid,
        in_specs=[pl.BlockSpec((_EDGE_BLK, 8), lambda i: (i, 0))] + wspec,
        out_specs=[pl.BlockSpec((_EDGE_BLK, 512), lambda i: (i, 0)),
                   pl.BlockSpec((_EDGE_BLK, 512), lambda i: (i, 0)),
                   pl.BlockSpec((_EDGE_BLK, 16), lambda i: (i, 0))],
        out_shape=[jax.ShapeDtypeStruct((E, 512), jnp.float32),
                   jax.ShapeDtypeStruct((E, 512), jnp.float32),
                   jax.ShapeDtypeStruct((E, 16), jnp.float32)],
    )(ea5, *rmlp1, *rmlp2)


def _perm_last_weight(w):
    return w.reshape(64, NCH, 4).transpose(0, 2, 1).reshape(64, NCH * 4)


def _conv_sorted(h0, shk, tpw, src, dst):
    hs = h0[src]
    cols = []
    for j in range(16):
        l = L_LIST[j]
        ht = hs * tpw[:, l * NCH:(l + 1) * NCH]
        cols.append(jax.ops.segment_sum(ht * shk[:, j:j + 1], dst,
                                        num_segments=N_NODES))
    return jnp.stack(cols, axis=-1)


def _product(A, sc0, sc1, one_hot, Wc0, Wc1, Wp0, Wp1):
    A0 = A[:, :, 0]
    A1 = A[:, :, 1:4]
    inv2 = jnp.sum(A * A, axis=-1)
    wc0 = jnp.einsum('ns,sck->nck', one_hot, Wc0)
    b0 = wc0[..., 0] * A0 + wc0[..., 1] * inv2 + wc0[..., 2] * (A0 * inv2)
    wc1 = jnp.einsum('ns,sck->nck', one_hot, Wc1)
    b1 = wc1[..., 0:1] * A1 + wc1[..., 1:2] * (A0[:, :, None] * A1)
    o0 = b0 @ Wp0 + sc0
    o1 = jnp.einsum('ncm,cd->ndm', b1, Wp1)
    if sc1 is not None:
        o1 = o1 + sc1
    return o0, o1


def kernel(edge_attr, edge_index, node_one_hot, params):
    src = edge_index[0]
    dst = edge_index[1]
    r0 = edge_attr[:, 0]
    keep = jnp.logical_not(
        jnp.logical_and(jnp.abs(r0) < 1e-7, src == dst)).astype(jnp.float32)

    sdst, ssrc, sr, sx, sy, sz, skeep = lax.sort(
        (dst, src, r0, edge_attr[:, 1], edge_attr[:, 2], edge_attr[:, 3], keep),
        num_keys=1)

    ea5 = jnp.stack([sr, sx, sy, sz, skeep,
                     jnp.zeros_like(sr), jnp.zeros_like(sr), jnp.zeros_like(sr)],
                    axis=1)

    rmlp1 = list(params['rmlp1'][:-1]) + [_perm_last_weight(params['rmlp1'][-1])]
    rmlp2 = list(params['rmlp2'][:-1]) + [_perm_last_weight(params['rmlp2'][-1])]
    tpw1, tpw2, shk = _edge_features(ea5, rmlp1, rmlp2)

    n = node_one_hot.shape[0]
    h = node_one_hot @ params['W_embed']
    h_up = h @ params['W_up1']
    up = h_up

    A = _conv_sorted(h_up, shk, tpw1, ssrc, sdst)
    WL = params['W_post1'][jnp.array(L_LIST)]
    A = jnp.einsum('ncj,jcd->ndj', A, WL)
    sc0 = jnp.einsum('nc,ns,scd->nd', h_up, node_one_hot, params['W_sc1'])
    h0, h1 = _product(A, sc0, None, node_one_hot, params['Wc1_0'],
                      params['Wc1_1'], params['Wp1_0'], params['Wp1_1'])
    out1 = jnp.concatenate([h0, h1.reshape(n, -1)], axis=-1)

    h0u = h0 @ params['W_up2_0']
    h1u = jnp.einsum('ncm,cd->ndm', h1, params['W_up2_1'])
    A2 = _conv_sorted(h0u, shk, tpw2, ssrc, sdst)
    WL2 = params['W_post2'][jnp.array(L_LIST)]
    A2 = jnp.einsum('ncj,jcd->ndj', A2, WL2)
    sc20 = jnp.einsum('nc,ns,scd->nd', h0u, node_one_hot, params['W_sc2_0'])
    sc21 = jnp.einsum('ncm,ns,scd->ndm', h1u, node_one_hot, params['W_sc2_1'])
    g0, g1 = _product(A2, sc20, sc21, node_one_hot, params['Wc2_0'],
                      params['Wc2_1'], params['Wp2_0'], params['Wp2_1'])
    out2 = jnp.concatenate([g0, g1.reshape(n, -1)], axis=-1)
    return out1, out2, up
